# Initial kernel scaffold; baseline (speedup 1.0000x reference)
import jax
import jax.numpy as jnp
from jax.experimental import pallas as pl


def kernel(x, weight, lora_a, lora_b):
    raise NotImplementedError("write your pallas kernel here")


# speedup vs baseline: 1.7139x; 1.7139x over previous
import functools

import jax
import jax.numpy as jnp
from jax import lax
from jax.experimental import pallas as pl
from jax.experimental.pallas import tpu as pltpu
from jax.experimental.pallas import tpu_sc as plsc

_SCALING = 1.0

_NC = 2
_NS = 16
_NW = _NC * _NS
_LANES = 16


def _sc_gather(n_tok, V, D, R, chunk):
  tpw = n_tok // _NW
  n_chunks = tpw // chunk
  mesh = plsc.VectorSubcoreMesh(core_axis_name="c", subcore_axis_name="s")

  @functools.partial(
      pl.kernel,
      mesh=mesh,
      compiler_params=pltpu.CompilerParams(use_tc_tiling_on_sc=False),
      out_type=(
          jax.ShapeDtypeStruct((n_tok, D), jnp.float32),
          jax.ShapeDtypeStruct((R, n_tok), jnp.float32),
      ),
      scratch_types=[
          pltpu.VMEM((chunk,), jnp.int32),
          pltpu.VMEM((chunk,), jnp.int32),
          pltpu.VMEM((chunk, D), jnp.float32),
          pltpu.VMEM((R, chunk), jnp.float32),
          pltpu.SemaphoreType.DMA,
          pltpu.SemaphoreType.DMA,
      ],
  )
  def k(xf_hbm, w_hbm, af_hbm, base_hbm, aT_hbm,
        idx_v, idx2_v, rows_v, acol_v, sem_w, sem_a):
    wid = lax.axis_index("s") * _NC + lax.axis_index("c")
    start = wid * tpw

    def body(ci, carry):
      off = start + ci * chunk
      pltpu.sync_copy(xf_hbm.at[pl.ds(off, chunk)], idx_v)
      pltpu.async_copy(w_hbm.at[idx_v], rows_v, sem_w).wait()
      pltpu.sync_copy(rows_v, base_hbm.at[pl.ds(off, chunk)])
      for r in range(R):
        for j in range(chunk // _LANES):
          sl = pl.ds(j * _LANES, _LANES)
          idx2_v[sl] = idx_v[sl] + jnp.int32(r * V)
        pltpu.async_copy(af_hbm.at[idx2_v], acol_v.at[r], sem_a).wait()
      pltpu.sync_copy(acol_v, aT_hbm.at[:, pl.ds(off, chunk)])
      return carry

    lax.fori_loop(0, n_chunks, body, 0)

  return k


def _tc_combine(n_tok, D, R, cb):
  def body(base_ref, aT_ref, b_ref, out_ref):
    delta = lax.dot_general(
        aT_ref[...], b_ref[...],
        (((0,), (1,)), ((), ())),
        preferred_element_type=jnp.float32,
    )
    out_ref[...] = base_ref[...] + delta * _SCALING

  return pl.pallas_call(
      body,
      grid=(n_tok // cb,),
      in_specs=[
          pl.BlockSpec((cb, D), lambda i: (i, 0)),
          pl.BlockSpec((R, cb), lambda i: (0, i)),
          pl.BlockSpec((D, R), lambda i: (0, 0)),
      ],
      out_specs=pl.BlockSpec((cb, D), lambda i: (i, 0)),
      out_shape=jax.ShapeDtypeStruct((n_tok, D), jnp.float32),
  )


@jax.jit
def kernel(x, weight, lora_a, lora_b):
  B, L = x.shape
  V, D = weight.shape
  R = lora_a.shape[0]
  n_tok = B * L

  xf = x.reshape(n_tok).astype(jnp.int32)
  af = lora_a.reshape(R * V)

  base, aT = _sc_gather(n_tok, V, D, R, chunk=128)(xf, weight, af)
  out = _tc_combine(n_tok, D, R, cb=2048)(base, aT, lora_b)
  return out.reshape(B, L, D)


# speedup vs baseline: 2.1344x; 1.2453x over previous
import functools

import jax
import jax.numpy as jnp
from jax import lax
from jax.experimental import pallas as pl
from jax.experimental.pallas import tpu as pltpu
from jax.experimental.pallas import tpu_sc as plsc

_SCALING = 1.0

_NC = 2
_NS = 16
_NW = _NC * _NS
_LANES = 16


def _sc_gather(n_tok, V, D, R, chunk):
  tpw = n_tok // _NW
  n_chunks = tpw // chunk
  nsub = chunk // 128
  mesh = plsc.VectorSubcoreMesh(core_axis_name="c", subcore_axis_name="s")

  @functools.partial(
      pl.kernel,
      mesh=mesh,
      compiler_params=pltpu.CompilerParams(use_tc_tiling_on_sc=False),
      out_type=(
          jax.ShapeDtypeStruct((n_tok, D), jnp.float32),
          jax.ShapeDtypeStruct((R, n_tok), jnp.float32),
      ),
      scratch_types=[
          pltpu.VMEM((nsub, 128), jnp.int32),
          pltpu.VMEM((R * nsub, 128), jnp.int32),
          pltpu.VMEM((chunk, D), jnp.float32),
          pltpu.VMEM((R, chunk), jnp.float32),
          pltpu.SemaphoreType.DMA,
          pltpu.SemaphoreType.DMA,
      ],
  )
  def k(xf_hbm, w_hbm, af_hbm, base_hbm, aT_hbm,
        idx_v, idx2_v, rows_v, acol_v, sem_w, sem_a):
    wid = lax.axis_index("s") * _NC + lax.axis_index("c")
    start = wid * tpw

    def body(ci, carry):
      off = start + ci * chunk
      pltpu.sync_copy(xf_hbm.at[pl.ds(off // 128, nsub)], idx_v)
      cps = [
          pltpu.async_copy(
              w_hbm.at[idx_v.at[j]],
              rows_v.at[pl.ds(j * 128, 128)],
              sem_w,
          )
          for j in range(nsub)
      ]
      for r in range(R):
        for j in range(nsub):
          for kk in range(128 // _LANES):
            sl = pl.ds(kk * _LANES, _LANES)
            idx2_v[r * nsub + j, sl] = idx_v[j, sl] + jnp.int32(r * V)
      cpa = [
          pltpu.async_copy(
              af_hbm.at[idx2_v.at[r * nsub + j]],
              acol_v.at[r, pl.ds(j * 128, 128)],
              sem_a,
          )
          for r in range(R)
          for j in range(nsub)
      ]
      for cp in cps:
        cp.wait()
      pltpu.sync_copy(rows_v, base_hbm.at[pl.ds(off, chunk)])
      for cp in cpa:
        cp.wait()
      pltpu.sync_copy(acol_v, aT_hbm.at[:, pl.ds(off, chunk)])
      return carry

    lax.fori_loop(0, n_chunks, body, 0)

  return k


def _tc_combine(n_tok, D, R, cb):
  def body(base_ref, aT_ref, b_ref, out_ref):
    delta = lax.dot_general(
        aT_ref[...], b_ref[...],
        (((0,), (1,)), ((), ())),
        preferred_element_type=jnp.float32,
    )
    out_ref[...] = base_ref[...] + delta * _SCALING

  return pl.pallas_call(
      body,
      grid=(n_tok // cb,),
      in_specs=[
          pl.BlockSpec((cb, D), lambda i: (i, 0)),
          pl.BlockSpec((R, cb), lambda i: (0, i)),
          pl.BlockSpec((D, R), lambda i: (0, 0)),
      ],
      out_specs=pl.BlockSpec((cb, D), lambda i: (i, 0)),
      out_shape=jax.ShapeDtypeStruct((n_tok, D), jnp.float32),
  )


@jax.jit
def kernel(x, weight, lora_a, lora_b):
  B, L = x.shape
  V, D = weight.shape
  R = lora_a.shape[0]
  n_tok = B * L

  xf = x.reshape(n_tok // 128, 128).astype(jnp.int32)
  af = lora_a.reshape(R * V)

  base, aT = _sc_gather(n_tok, V, D, R, chunk=640)(xf, weight, af)
  out = _tc_combine(n_tok, D, R, cb=2048)(base, aT, lora_b)
  return out.reshape(B, L, D)


# speedup vs baseline: 2.1613x; 1.0126x over previous
import functools

import jax
import jax.numpy as jnp
from jax import lax
from jax.experimental import pallas as pl
from jax.experimental.pallas import tpu as pltpu
from jax.experimental.pallas import tpu_sc as plsc

_SCALING = 1.0

_NC = 2
_NS = 16
_NW = _NC * _NS


def _sc_gather(n_tok, V, D, R, chunk):
  tpw = n_tok // _NW
  n_chunks = tpw // chunk
  nsub = chunk // 128
  mesh = plsc.VectorSubcoreMesh(core_axis_name="c", subcore_axis_name="s")

  @functools.partial(
      pl.kernel,
      mesh=mesh,
      compiler_params=pltpu.CompilerParams(use_tc_tiling_on_sc=False),
      out_type=(
          jax.ShapeDtypeStruct((n_tok, D), jnp.float32),
          jax.ShapeDtypeStruct((R, n_tok), jnp.float32),
      ),
      scratch_types=[
          pltpu.VMEM((nsub, 128), jnp.int32),
          pltpu.VMEM((chunk, D), jnp.float32),
          pltpu.VMEM((R, chunk), jnp.float32),
          pltpu.SemaphoreType.DMA,
          pltpu.SemaphoreType.DMA,
      ],
  )
  def k(xf_hbm, w_hbm, a_hbm, base_hbm, aT_hbm,
        idx_v, rows_v, acol_v, sem_w, sem_a):
    wid = lax.axis_index("s") * _NC + lax.axis_index("c")
    start = wid * tpw

    def body(ci, carry):
      off = start + ci * chunk
      pltpu.sync_copy(xf_hbm.at[pl.ds(off // 128, nsub)], idx_v)
      cps = [
          pltpu.async_copy(
              w_hbm.at[idx_v.at[j]],
              rows_v.at[pl.ds(j * 128, 128)],
              sem_w,
          )
          for j in range(nsub)
      ]
      cpa = [
          pltpu.async_copy(
              a_hbm.at[r].at[idx_v.at[j]],
              acol_v.at[r, pl.ds(j * 128, 128)],
              sem_a,
          )
          for r in range(R)
          for j in range(nsub)
      ]
      for cp in cps:
        cp.wait()
      pltpu.sync_copy(rows_v, base_hbm.at[pl.ds(off, chunk)])
      for cp in cpa:
        cp.wait()
      pltpu.sync_copy(acol_v, aT_hbm.at[:, pl.ds(off, chunk)])
      return carry

    lax.fori_loop(0, n_chunks, body, 0)

  return k


def _tc_combine(n_tok, L, D, R, bc):
  B = n_tok // L
  nb = B // bc

  def body(base_ref, aT_ref, b_ref, out_ref):
    delta = lax.dot_general(
        b_ref[...], aT_ref[...],
        (((1,), (0,)), ((), ())),
        preferred_element_type=jnp.float32,
    )
    out_ref[...] = base_ref[...].T + delta * _SCALING

  return pl.pallas_call(
      body,
      grid=(L, nb),
      in_specs=[
          pl.BlockSpec((bc, D), lambda l, j: (l * nb + j, 0)),
          pl.BlockSpec((R, bc), lambda l, j: (0, l * nb + j)),
          pl.BlockSpec((D, R), lambda l, j: (0, 0)),
      ],
      out_specs=pl.BlockSpec((D, bc), lambda l, j: (l, j)),
      out_shape=jax.ShapeDtypeStruct((L * D, B), jnp.float32),
  )


@jax.jit
def kernel(x, weight, lora_a, lora_b):
  B, L = x.shape
  V, D = weight.shape
  R = lora_a.shape[0]
  n_tok = B * L

  xf = x.T.reshape(n_tok // 128, 128).astype(jnp.int32)

  base, aT = _sc_gather(n_tok, V, D, R, chunk=1280)(xf, weight, lora_a)
  out2 = _tc_combine(n_tok, L, D, R, bc=512)(base, aT, lora_b)
  return out2.reshape(L, D, B).transpose(2, 0, 1)


# speedup vs baseline: 5.0268x; 2.3259x over previous
import functools

import jax
import jax.numpy as jnp
from jax import lax
from jax.experimental import pallas as pl
from jax.experimental.pallas import tpu as pltpu
from jax.experimental.pallas import tpu_sc as plsc

_SCALING = 1.0

_NC = 2
_NS = 16
_NW = _NC * _NS


def _tc_prep(V, D, R, vb):
  pad = 128 - D - R
  n_blk = (V + vb - 1) // vb

  def body(wT_ref, a_ref, t_ref):
    wblk = wT_ref[...].T
    ablk = a_ref[...].T
    t_ref[...] = jnp.concatenate(
        [wblk, ablk * _SCALING, jnp.zeros((vb, pad), jnp.float32)], axis=1)

  return pl.pallas_call(
      body,
      grid=(n_blk,),
      in_specs=[
          pl.BlockSpec((D, vb), lambda i: (0, i)),
          pl.BlockSpec((R, vb), lambda i: (0, i)),
      ],
      out_specs=pl.BlockSpec((vb, 128), lambda i: (i, 0)),
      out_shape=jax.ShapeDtypeStruct((V, 128), jnp.float32),
  )


def _sc_gather(n_tok, V, chunk):
  tpw = n_tok // _NW
  n_chunks = tpw // chunk
  nsub = chunk // 128
  mesh = plsc.VectorSubcoreMesh(core_axis_name="c", subcore_axis_name="s")

  @functools.partial(
      pl.kernel,
      mesh=mesh,
      compiler_params=pltpu.CompilerParams(use_tc_tiling_on_sc=False),
      out_type=jax.ShapeDtypeStruct((n_tok, 128), jnp.float32),
      scratch_types=[
          pltpu.VMEM((nsub, 128), jnp.int32),
          pltpu.VMEM((chunk, 128), jnp.float32),
          pltpu.SemaphoreType.DMA,
      ],
  )
  def k(xf_hbm, t_hbm, g_hbm, idx_v, rows_v, sem):
    wid = lax.axis_index("s") * _NC + lax.axis_index("c")
    start = wid * tpw

    def body(ci, carry):
      off = start + ci * chunk
      pltpu.sync_copy(xf_hbm.at[pl.ds(off // 128, nsub)], idx_v)
      cps = [
          pltpu.async_copy(
              t_hbm.at[idx_v.at[j]],
              rows_v.at[pl.ds(j * 128, 128)],
              sem,
          )
          for j in range(nsub)
      ]
      for cp in cps:
        cp.wait()
      pltpu.sync_copy(rows_v, g_hbm.at[pl.ds(off, chunk)])
      return carry

    lax.fori_loop(0, n_chunks, body, 0)

  return k


def _tc_combine(n_tok, L, D, bc):
  B = n_tok // L
  nb = B // bc

  def body(g_ref, m_ref, out_ref):
    out_ref[...] = lax.dot_general(
        m_ref[...], g_ref[...],
        (((1,), (1,)), ((), ())),
        preferred_element_type=jnp.float32,
    )

  return pl.pallas_call(
      body,
      grid=(L, nb),
      in_specs=[
          pl.BlockSpec((bc, 128), lambda l, j: (l * nb + j, 0)),
          pl.BlockSpec((D, 128), lambda l, j: (0, 0)),
      ],
      out_specs=pl.BlockSpec((D, bc), lambda l, j: (l, j)),
      out_shape=jax.ShapeDtypeStruct((L * D, B), jnp.float32),
  )


@jax.jit
def kernel(x, weight, lora_a, lora_b):
  B, L = x.shape
  V, D = weight.shape
  R = lora_a.shape[0]
  n_tok = B * L

  table = _tc_prep(V, D, R, vb=2048)(weight.T, lora_a)

  xf = x.T.reshape(n_tok // 128, 128).astype(jnp.int32)
  g = _sc_gather(n_tok, V, chunk=640)(xf, table)

  m = jnp.concatenate(
      [jnp.eye(D, dtype=jnp.float32), lora_b,
       jnp.zeros((D, 128 - D - R), jnp.float32)], axis=1)
  out2 = _tc_combine(n_tok, L, D, bc=512)(g, m)
  return out2.reshape(L, D, B).transpose(2, 0, 1)


# speedup vs baseline: 5.6048x; 1.1150x over previous
import functools

import jax
import jax.numpy as jnp
from jax import lax
from jax.experimental import pallas as pl
from jax.experimental.pallas import tpu as pltpu
from jax.experimental.pallas import tpu_sc as plsc

_SCALING = 1.0

_NC = 2
_NS = 16
_NW = _NC * _NS

_SB = 2048


def _tc_prep(V, D, R):
  pad = 128 - D - 2 * R
  n_blk = (V + _SB - 1) // _SB
  hb = _SB // 2

  def pack(wT_ref, a_ref):
    w = wT_ref[...].T
    a = a_ref[...].T
    lo = w.astype(jnp.bfloat16)
    hi = jnp.concatenate(
        [(a * _SCALING).astype(jnp.bfloat16),
         jnp.zeros((hb, D - R), jnp.bfloat16)], axis=1)
    lo_u = lax.bitcast_convert_type(lo, jnp.uint16).astype(jnp.uint32)
    hi_u = lax.bitcast_convert_type(hi, jnp.uint16).astype(jnp.uint32)
    return lax.bitcast_convert_type(lo_u | (hi_u << 16), jnp.int32)

  def body(wT1_ref, a1_ref, wT2_ref, a2_ref, t_ref):
    t_ref[...] = jnp.concatenate(
        [pack(wT1_ref, a1_ref), pack(wT2_ref, a2_ref)], axis=1)

  return pl.pallas_call(
      body,
      grid=(n_blk,),
      in_specs=[
          pl.BlockSpec((D, hb), lambda i: (0, 2 * i)),
          pl.BlockSpec((R, hb), lambda i: (0, 2 * i)),
          pl.BlockSpec((D, hb), lambda i: (0, jnp.minimum(2 * i + 1, V // hb))),
          pl.BlockSpec((R, hb), lambda i: (0, jnp.minimum(2 * i + 1, V // hb))),
      ],
      out_specs=pl.BlockSpec((hb, 128), lambda i: (i, 0)),
      out_shape=jax.ShapeDtypeStruct((n_blk * hb, 128), jnp.int32),
  ), n_blk * _SB


def _sc_gather(n_tok, Vp, chunk):
  tpw = n_tok // _NW
  n_chunks = tpw // chunk
  nsub = chunk // 128
  mesh = plsc.VectorSubcoreMesh(core_axis_name="c", subcore_axis_name="s")

  @functools.partial(
      pl.kernel,
      mesh=mesh,
      compiler_params=pltpu.CompilerParams(use_tc_tiling_on_sc=False),
      out_type=jax.ShapeDtypeStruct((n_tok, 64), jnp.int32),
      scratch_types=[
          pltpu.VMEM((nsub, 128), jnp.int32),
          pltpu.VMEM((nsub, 128), jnp.int32),
          pltpu.VMEM((chunk, 64), jnp.int32),
          pltpu.SemaphoreType.DMA,
      ],
  )
  def k(xf_hbm, t_hbm, g_hbm, idx_v, idx2_v, rows_v, sem):
    wid = lax.axis_index("s") * _NC + lax.axis_index("c")
    start = wid * tpw

    def body(ci, carry):
      off = start + ci * chunk
      pltpu.sync_copy(xf_hbm.at[pl.ds(off // 128, nsub)], idx_v)
      for j in range(nsub):
        for kk in range(8):
          sl = pl.ds(kk * 16, 16)
          v = idx_v[j, sl]
          idx2_v[j, sl] = (
              (v >> 11) << 11
          ) + ((v & 1023) << 1) + ((v >> 10) & 1)
      cps = [
          pltpu.async_copy(
              t_hbm.at[idx2_v.at[j]],
              rows_v.at[pl.ds(j * 128, 128)],
              sem,
          )
          for j in range(nsub)
      ]
      for cp in cps:
        cp.wait()
      pltpu.sync_copy(rows_v, g_hbm.at[pl.ds(off, chunk)])
      return carry

    lax.fori_loop(0, n_chunks, body, 0)

  return k


def _tc_combine(n_tok, L, D, bc):
  B = n_tok // L
  nb = B // bc
  hc = bc // 2

  def body(g_ref, mlo_ref, mhi_ref, out_ref):
    gu = lax.bitcast_convert_type(g_ref[...], jnp.uint32)
    lo = lax.bitcast_convert_type(
        (gu & 0xFFFF).astype(jnp.uint16), jnp.bfloat16)
    hi = lax.bitcast_convert_type(
        (gu >> 16).astype(jnp.uint16), jnp.bfloat16)
    dims = (((1,), (1,)), ((), ()))
    out_ref[:, :hc] = lax.dot_general(
        mlo_ref[...], lo[:, :64], dims, preferred_element_type=jnp.float32
    ) + lax.dot_general(
        mhi_ref[...], hi[:, :64], dims, preferred_element_type=jnp.float32)
    out_ref[:, hc:] = lax.dot_general(
        mlo_ref[...], lo[:, 64:], dims, preferred_element_type=jnp.float32
    ) + lax.dot_general(
        mhi_ref[...], hi[:, 64:], dims, preferred_element_type=jnp.float32)

  return pl.pallas_call(
      body,
      grid=(L, nb),
      in_specs=[
          pl.BlockSpec((hc, 128), lambda l, j: (l * nb + j, 0)),
          pl.BlockSpec((D, D), lambda l, j: (0, 0)),
          pl.BlockSpec((D, D), lambda l, j: (0, 0)),
      ],
      out_specs=pl.BlockSpec((D, bc), lambda l, j: (l, j)),
      out_shape=jax.ShapeDtypeStruct((L * D, B), jnp.float32),
  )


@jax.jit
def kernel(x, weight, lora_a, lora_b):
  B, L = x.shape
  V, D = weight.shape
  R = lora_a.shape[0]
  n_tok = B * L
  bc = 512
  hc = bc // 2

  prep, Vp = _tc_prep(V, D, R)
  wT = weight.T
  table = prep(wT, lora_a, wT, lora_a)
  t64 = table.reshape(Vp, 64)

  xp = (x.T.astype(jnp.int32)
        .reshape(L, B // bc, 2, hc)
        .transpose(0, 1, 3, 2)
        .reshape(n_tok // 128, 128))
  g = _sc_gather(n_tok, Vp, chunk=640)(xp, t64)
  g2 = g.reshape(n_tok // 2, 128)

  mlo = jnp.eye(D, dtype=jnp.bfloat16)
  mhi = jnp.concatenate(
      [lora_b, jnp.zeros((D, D - R), jnp.float32)], axis=1).astype(jnp.bfloat16)
  out2 = _tc_combine(n_tok, L, D, bc=bc)(g2, mlo, mhi)
  return out2.reshape(L, D, B).transpose(2, 0, 1)


# speedup vs baseline: 7.8325x; 1.3975x over previous
import functools

import jax
import jax.numpy as jnp
from jax import lax
from jax.experimental import pallas as pl
from jax.experimental.pallas import tpu as pltpu
from jax.experimental.pallas import tpu_sc as plsc

_SCALING = 1.0

_NC = 2
_NS = 16
_NW = _NC * _NS

_SB = 4096


def _tc_prep(V, D, R):
  pad = 128 - D - 2 * R
  n_blk = (V + _SB - 1) // _SB
  hb = _SB // 2

  def pack(wT_ref, a_ref):
    w = wT_ref[...].T
    a = a_ref[...].T
    lo = w.astype(jnp.bfloat16)
    hi = jnp.concatenate(
        [(a * _SCALING).astype(jnp.bfloat16),
         jnp.zeros((hb, D - R), jnp.bfloat16)], axis=1)
    lo_u = lax.bitcast_convert_type(lo, jnp.uint16).astype(jnp.uint32)
    hi_u = lax.bitcast_convert_type(hi, jnp.uint16).astype(jnp.uint32)
    return lax.bitcast_convert_type(lo_u | (hi_u << 16), jnp.int32)

  def body(wT1_ref, a1_ref, wT2_ref, a2_ref, t_ref):
    t_ref[...] = jnp.concatenate(
        [pack(wT1_ref, a1_ref), pack(wT2_ref, a2_ref)], axis=1)

  return pl.pallas_call(
      body,
      grid=(n_blk,),
      in_specs=[
          pl.BlockSpec((D, hb), lambda i: (0, 2 * i)),
          pl.BlockSpec((R, hb), lambda i: (0, 2 * i)),
          pl.BlockSpec((D, hb), lambda i: (0, jnp.minimum(2 * i + 1, V // hb))),
          pl.BlockSpec((R, hb), lambda i: (0, jnp.minimum(2 * i + 1, V // hb))),
      ],
      out_specs=pl.BlockSpec((hb, 128), lambda i: (i, 0)),
      out_shape=jax.ShapeDtypeStruct((n_blk * hb, 128), jnp.int32),
  ), n_blk * _SB


def _sc_gather(n_tok, Vp, chunk):
  tpw = n_tok // _NW
  n_chunks = tpw // chunk
  nsub = chunk // 128
  mesh = plsc.VectorSubcoreMesh(core_axis_name="c", subcore_axis_name="s")

  @functools.partial(
      pl.kernel,
      mesh=mesh,
      compiler_params=pltpu.CompilerParams(use_tc_tiling_on_sc=False),
      out_type=jax.ShapeDtypeStruct((n_tok, 64), jnp.int32),
      scratch_types=[
          pltpu.VMEM((nsub, 128), jnp.int32),
          pltpu.VMEM((nsub, 128), jnp.int32),
          pltpu.VMEM((chunk, 64), jnp.int32),
          pltpu.SemaphoreType.DMA,
      ],
  )
  def k(xf_hbm, t_hbm, g_hbm, idx_v, idx2_v, rows_v, sem):
    wid = lax.axis_index("s") * _NC + lax.axis_index("c")
    start = wid * tpw

    def body(ci, carry):
      off = start + ci * chunk
      pltpu.sync_copy(xf_hbm.at[pl.ds(off // 128, nsub)], idx_v)
      for j in range(nsub):
        for kk in range(8):
          sl = pl.ds(kk * 16, 16)
          v = idx_v[j, sl]
          idx2_v[j, sl] = (
              (v >> 12) << 12
          ) + ((v & 2047) << 1) + ((v >> 11) & 1)
      cps = [
          pltpu.async_copy(
              t_hbm.at[idx2_v.at[j]],
              rows_v.at[pl.ds(j * 128, 128)],
              sem,
          )
          for j in range(nsub)
      ]
      for cp in cps:
        cp.wait()
      pltpu.sync_copy(rows_v, g_hbm.at[pl.ds(off, chunk)])
      return carry

    lax.fori_loop(0, n_chunks, body, 0)

  return k


def _tc_combine(n_tok, L, D, bc):
  B = n_tok // L
  nb = B // bc
  hc = bc // 2

  def body(g_ref, mlo_ref, mhi_ref, out_ref):
    gu = lax.bitcast_convert_type(g_ref[...], jnp.uint32)
    lo = lax.bitcast_convert_type(
        (gu & 0xFFFF).astype(jnp.uint16), jnp.bfloat16)
    hi = lax.bitcast_convert_type(
        (gu >> 16).astype(jnp.uint16), jnp.bfloat16)
    dims = (((1,), (1,)), ((), ()))
    out_ref[:, :hc] = lax.dot_general(
        mlo_ref[...], lo[:, :64], dims, preferred_element_type=jnp.float32
    ) + lax.dot_general(
        mhi_ref[...], hi[:, :64], dims, preferred_element_type=jnp.float32)
    out_ref[:, hc:] = lax.dot_general(
        mlo_ref[...], lo[:, 64:], dims, preferred_element_type=jnp.float32
    ) + lax.dot_general(
        mhi_ref[...], hi[:, 64:], dims, preferred_element_type=jnp.float32)

  return pl.pallas_call(
      body,
      grid=(L, nb),
      in_specs=[
          pl.BlockSpec((hc, 128), lambda l, j: (l * nb + j, 0)),
          pl.BlockSpec((D, D), lambda l, j: (0, 0)),
          pl.BlockSpec((D, D), lambda l, j: (0, 0)),
      ],
      out_specs=pl.BlockSpec((D, bc), lambda l, j: (l, j)),
      out_shape=jax.ShapeDtypeStruct((L * D, B), jnp.float32),
  )


@jax.jit
def kernel(x, weight, lora_a, lora_b):
  B, L = x.shape
  V, D = weight.shape
  R = lora_a.shape[0]
  n_tok = B * L
  bc = 1024
  hc = bc // 2

  prep, Vp = _tc_prep(V, D, R)
  wT = weight.T
  table = prep(wT, lora_a, wT, lora_a)
  t64 = table.reshape(Vp, 64)

  xp = (x.T.astype(jnp.int32)
        .reshape(L, B // bc, 2, hc)
        .transpose(0, 1, 3, 2)
        .reshape(n_tok // 128, 128))
  g = _sc_gather(n_tok, Vp, chunk=1280)(xp, t64)
  g2 = g.reshape(n_tok // 2, 128)

  mlo = jnp.eye(D, dtype=jnp.bfloat16)
  mhi = jnp.concatenate(
      [lora_b, jnp.zeros((D, D - R), jnp.float32)], axis=1).astype(jnp.bfloat16)
  out2 = _tc_combine(n_tok, L, D, bc=bc)(g2, mlo, mhi)
  return out2.reshape(L, D, B).transpose(2, 0, 1)


# speedup vs baseline: 9.7250x; 1.2416x over previous
import functools

import jax
import jax.numpy as jnp
from jax import lax
from jax.experimental import pallas as pl
from jax.experimental.pallas import tpu as pltpu
from jax.experimental.pallas import tpu_sc as plsc

_SCALING = 1.0

_NC = 2
_NS = 16
_NW = _NC * _NS

_SB = 8192


def _tc_prep(V, D, R):
  pad = 128 - D - 2 * R
  n_blk = (V + _SB - 1) // _SB
  hb = _SB // 2

  def pack(wT_ref, a_ref):
    w = wT_ref[...].T
    a = a_ref[...].T
    lo = w.astype(jnp.bfloat16)
    hi = jnp.concatenate(
        [(a * _SCALING).astype(jnp.bfloat16),
         jnp.zeros((hb, D - R), jnp.bfloat16)], axis=1)
    lo_u = lax.bitcast_convert_type(lo, jnp.uint16).astype(jnp.uint32)
    hi_u = lax.bitcast_convert_type(hi, jnp.uint16).astype(jnp.uint32)
    return lax.bitcast_convert_type(lo_u | (hi_u << 16), jnp.int32)

  def body(wT1_ref, a1_ref, wT2_ref, a2_ref, t_ref):
    t_ref[...] = jnp.concatenate(
        [pack(wT1_ref, a1_ref), pack(wT2_ref, a2_ref)], axis=1)

  return pl.pallas_call(
      body,
      grid=(n_blk,),
      in_specs=[
          pl.BlockSpec((D, hb), lambda i: (0, 2 * i)),
          pl.BlockSpec((R, hb), lambda i: (0, 2 * i)),
          pl.BlockSpec((D, hb), lambda i: (0, jnp.minimum(2 * i + 1, V // hb))),
          pl.BlockSpec((R, hb), lambda i: (0, jnp.minimum(2 * i + 1, V // hb))),
      ],
      out_specs=pl.BlockSpec((hb, 128), lambda i: (i, 0)),
      out_shape=jax.ShapeDtypeStruct((n_blk * hb, 128), jnp.int32),
  ), n_blk * _SB


def _sc_gather(n_tok, Vp, chunk):
  tpw = n_tok // _NW
  n_chunks = tpw // chunk
  nsub = chunk // 128
  mesh = plsc.VectorSubcoreMesh(core_axis_name="c", subcore_axis_name="s")

  @functools.partial(
      pl.kernel,
      mesh=mesh,
      compiler_params=pltpu.CompilerParams(use_tc_tiling_on_sc=False),
      out_type=jax.ShapeDtypeStruct((n_tok, 64), jnp.int32),
      scratch_types=[
          pltpu.VMEM((nsub, 128), jnp.int32),
          pltpu.VMEM((nsub, 128), jnp.int32),
          pltpu.VMEM((chunk, 64), jnp.int32),
          pltpu.SemaphoreType.DMA,
      ],
  )
  def k(xf_hbm, t_hbm, g_hbm, idx_v, idx2_v, rows_v, sem):
    wid = lax.axis_index("s") * _NC + lax.axis_index("c")
    start = wid * tpw

    def body(ci, carry):
      off = start + ci * chunk
      pltpu.sync_copy(xf_hbm.at[pl.ds(off // 128, nsub)], idx_v)
      for j in range(nsub):
        for kk in range(8):
          sl = pl.ds(kk * 16, 16)
          v = idx_v[j, sl]
          idx2_v[j, sl] = (
              (v >> 13) << 13
          ) + ((v & 4095) << 1) + ((v >> 12) & 1)
      cps = [
          pltpu.async_copy(
              t_hbm.at[idx2_v.at[j]],
              rows_v.at[pl.ds(j * 128, 128)],
              sem,
          )
          for j in range(nsub)
      ]
      for cp in cps:
        cp.wait()
      pltpu.sync_copy(rows_v, g_hbm.at[pl.ds(off, chunk)])
      return carry

    lax.fori_loop(0, n_chunks, body, 0)

  return k


def _tc_combine(n_tok, L, D, bc):
  B = n_tok // L
  nb = B // bc
  hc = bc // 2

  def body(g_ref, mlo_ref, mhi_ref, out_ref):
    gu = lax.bitcast_convert_type(g_ref[...], jnp.uint32)
    lo = lax.bitcast_convert_type(
        (gu & 0xFFFF).astype(jnp.uint16), jnp.bfloat16)
    hi = lax.bitcast_convert_type(
        (gu >> 16).astype(jnp.uint16), jnp.bfloat16)
    dims = (((1,), (1,)), ((), ()))
    out_ref[:, :hc] = lax.dot_general(
        mlo_ref[...], lo[:, :64], dims, preferred_element_type=jnp.float32
    ) + lax.dot_general(
        mhi_ref[...], hi[:, :64], dims, preferred_element_type=jnp.float32)
    out_ref[:, hc:] = lax.dot_general(
        mlo_ref[...], lo[:, 64:], dims, preferred_element_type=jnp.float32
    ) + lax.dot_general(
        mhi_ref[...], hi[:, 64:], dims, preferred_element_type=jnp.float32)

  return pl.pallas_call(
      body,
      grid=(L, nb),
      in_specs=[
          pl.BlockSpec((hc, 128), lambda l, j: (l * nb + j, 0)),
          pl.BlockSpec((D, D), lambda l, j: (0, 0)),
          pl.BlockSpec((D, D), lambda l, j: (0, 0)),
      ],
      out_specs=pl.BlockSpec((D, bc), lambda l, j: (l, j)),
      out_shape=jax.ShapeDtypeStruct((L * D, B), jnp.float32),
  )


@jax.jit
def kernel(x, weight, lora_a, lora_b):
  B, L = x.shape
  V, D = weight.shape
  R = lora_a.shape[0]
  n_tok = B * L
  bc = 2048
  hc = bc // 2

  prep, Vp = _tc_prep(V, D, R)
  wT = weight.T
  table = prep(wT, lora_a, wT, lora_a)
  t64 = table.reshape(Vp, 64)

  xp = (x.T.astype(jnp.int32)
        .reshape(L, B // bc, 2, hc)
        .transpose(0, 1, 3, 2)
        .reshape(n_tok // 128, 128))
  g = _sc_gather(n_tok, Vp, chunk=1280)(xp, t64)
  g2 = g.reshape(n_tok // 2, 128)

  mlo = jnp.eye(D, dtype=jnp.bfloat16)
  mhi = jnp.concatenate(
      [lora_b, jnp.zeros((D, D - R), jnp.float32)], axis=1).astype(jnp.bfloat16)
  out2 = _tc_combine(n_tok, L, D, bc=bc)(g2, mlo, mhi)
  return out2.reshape(L, D, B).transpose(2, 0, 1)


# speedup vs baseline: 11.0420x; 1.1354x over previous
import functools

import jax
import jax.numpy as jnp
from jax import lax
from jax.experimental import pallas as pl
from jax.experimental.pallas import tpu as pltpu
from jax.experimental.pallas import tpu_sc as plsc

_SCALING = 1.0

_NC = 2
_NS = 16
_NW = _NC * _NS

_SB = 16384


def _tc_prep(V, D, R):
  pad = 128 - D - 2 * R
  n_blk = (V + _SB - 1) // _SB
  hb = _SB // 2

  def pack(wT_ref, a_ref):
    w = wT_ref[...].T
    a = a_ref[...].T
    lo = w.astype(jnp.bfloat16)
    hi = jnp.concatenate(
        [(a * _SCALING).astype(jnp.bfloat16),
         jnp.zeros((hb, D - R), jnp.bfloat16)], axis=1)
    lo_u = lax.bitcast_convert_type(lo, jnp.uint16).astype(jnp.uint32)
    hi_u = lax.bitcast_convert_type(hi, jnp.uint16).astype(jnp.uint32)
    return lax.bitcast_convert_type(lo_u | (hi_u << 16), jnp.int32)

  def body(wT1_ref, a1_ref, wT2_ref, a2_ref, t_ref):
    t_ref[...] = jnp.concatenate(
        [pack(wT1_ref, a1_ref), pack(wT2_ref, a2_ref)], axis=1)

  return pl.pallas_call(
      body,
      grid=(n_blk,),
      in_specs=[
          pl.BlockSpec((D, hb), lambda i: (0, 2 * i)),
          pl.BlockSpec((R, hb), lambda i: (0, 2 * i)),
          pl.BlockSpec((D, hb), lambda i: (0, jnp.minimum(2 * i + 1, V // hb))),
          pl.BlockSpec((R, hb), lambda i: (0, jnp.minimum(2 * i + 1, V // hb))),
      ],
      out_specs=pl.BlockSpec((hb, 128), lambda i: (i, 0)),
      out_shape=jax.ShapeDtypeStruct((n_blk * hb, 128), jnp.int32),
  ), n_blk * _SB


def _sc_gather(n_tok, Vp, chunk):
  tpw = n_tok // _NW
  n_chunks = tpw // chunk
  nsub = chunk // 128
  mesh = plsc.VectorSubcoreMesh(core_axis_name="c", subcore_axis_name="s")

  @functools.partial(
      pl.kernel,
      mesh=mesh,
      compiler_params=pltpu.CompilerParams(use_tc_tiling_on_sc=False),
      out_type=jax.ShapeDtypeStruct((n_tok, 64), jnp.int32),
      scratch_types=[
          pltpu.VMEM((nsub, 128), jnp.int32),
          pltpu.VMEM((nsub, 128), jnp.int32),
          pltpu.VMEM((chunk, 64), jnp.int32),
          pltpu.SemaphoreType.DMA,
      ],
  )
  def k(xf_hbm, t_hbm, g_hbm, idx_v, idx2_v, rows_v, sem):
    wid = lax.axis_index("s") * _NC + lax.axis_index("c")
    start = wid * tpw

    def body(ci, carry):
      off = start + ci * chunk
      pltpu.sync_copy(xf_hbm.at[pl.ds(off // 128, nsub)], idx_v)
      for j in range(nsub):
        for kk in range(8):
          sl = pl.ds(kk * 16, 16)
          v = idx_v[j, sl]
          idx2_v[j, sl] = (
              (v >> 14) << 14
          ) + ((v & 8191) << 1) + ((v >> 13) & 1)
      cps = [
          pltpu.async_copy(
              t_hbm.at[idx2_v.at[j]],
              rows_v.at[pl.ds(j * 128, 128)],
              sem,
          )
          for j in range(nsub)
      ]
      for cp in cps:
        cp.wait()
      pltpu.sync_copy(rows_v, g_hbm.at[pl.ds(off, chunk)])
      return carry

    lax.fori_loop(0, n_chunks, body, 0)

  return k


def _tc_combine(n_tok, L, D, bc):
  B = n_tok // L
  nb = B // bc
  hc = bc // 2

  def body(g_ref, mlo_ref, mhi_ref, out_ref):
    gu = lax.bitcast_convert_type(g_ref[...], jnp.uint32)
    lo = lax.bitcast_convert_type(
        (gu & 0xFFFF).astype(jnp.uint16), jnp.bfloat16)
    hi = lax.bitcast_convert_type(
        (gu >> 16).astype(jnp.uint16), jnp.bfloat16)
    dims = (((1,), (1,)), ((), ()))
    out_ref[:, :hc] = lax.dot_general(
        mlo_ref[...], lo[:, :64], dims, preferred_element_type=jnp.float32
    ) + lax.dot_general(
        mhi_ref[...], hi[:, :64], dims, preferred_element_type=jnp.float32)
    out_ref[:, hc:] = lax.dot_general(
        mlo_ref[...], lo[:, 64:], dims, preferred_element_type=jnp.float32
    ) + lax.dot_general(
        mhi_ref[...], hi[:, 64:], dims, preferred_element_type=jnp.float32)

  return pl.pallas_call(
      body,
      grid=(L, nb),
      in_specs=[
          pl.BlockSpec((hc, 128), lambda l, j: (l * nb + j, 0)),
          pl.BlockSpec((D, D), lambda l, j: (0, 0)),
          pl.BlockSpec((D, D), lambda l, j: (0, 0)),
      ],
      out_specs=pl.BlockSpec((D, bc), lambda l, j: (l, j)),
      out_shape=jax.ShapeDtypeStruct((L * D, B), jnp.float32),
  )


@jax.jit
def kernel(x, weight, lora_a, lora_b):
  B, L = x.shape
  V, D = weight.shape
  R = lora_a.shape[0]
  n_tok = B * L
  bc = 4096
  hc = bc // 2

  prep, Vp = _tc_prep(V, D, R)
  wT = weight.T
  table = prep(wT, lora_a, wT, lora_a)
  t64 = table.reshape(Vp, 64)

  xp = (x.T.astype(jnp.int32)
        .reshape(L, B // bc, 2, hc)
        .transpose(0, 1, 3, 2)
        .reshape(n_tok // 128, 128))
  g = _sc_gather(n_tok, Vp, chunk=1280)(xp, t64)
  g2 = g.reshape(n_tok // 2, 128)

  mlo = jnp.eye(D, dtype=jnp.bfloat16)
  mhi = jnp.concatenate(
      [lora_b, jnp.zeros((D, D - R), jnp.float32)], axis=1).astype(jnp.bfloat16)
  out2 = _tc_combine(n_tok, L, D, bc=bc)(g2, mlo, mhi)
  return out2.reshape(L, D, B).transpose(2, 0, 1)


# speedup vs baseline: 11.6070x; 1.0512x over previous
import functools

import jax
import jax.numpy as jnp
from jax import lax
from jax.experimental import pallas as pl
from jax.experimental.pallas import tpu as pltpu
from jax.experimental.pallas import tpu_sc as plsc

_SCALING = 1.0

_NC = 2
_NS = 16
_NW = _NC * _NS

_SB = 32768


def _tc_prep(V, D, R):
  pad = 128 - D - 2 * R
  n_blk = (V + _SB - 1) // _SB
  hb = _SB // 2

  def pack(wT_ref, a_ref):
    w = wT_ref[...].T
    a = a_ref[...].T
    lo = w.astype(jnp.bfloat16)
    hi = jnp.concatenate(
        [(a * _SCALING).astype(jnp.bfloat16),
         jnp.zeros((hb, D - R), jnp.bfloat16)], axis=1)
    lo_u = lax.bitcast_convert_type(lo, jnp.uint16).astype(jnp.uint32)
    hi_u = lax.bitcast_convert_type(hi, jnp.uint16).astype(jnp.uint32)
    return lax.bitcast_convert_type(lo_u | (hi_u << 16), jnp.int32)

  def body(wT1_ref, a1_ref, wT2_ref, a2_ref, t_ref):
    t_ref[...] = jnp.concatenate(
        [pack(wT1_ref, a1_ref), pack(wT2_ref, a2_ref)], axis=1)

  return pl.pallas_call(
      body,
      grid=(n_blk,),
      in_specs=[
          pl.BlockSpec((D, hb), lambda i: (0, 2 * i)),
          pl.BlockSpec((R, hb), lambda i: (0, 2 * i)),
          pl.BlockSpec((D, hb), lambda i: (0, jnp.minimum(2 * i + 1, V // hb))),
          pl.BlockSpec((R, hb), lambda i: (0, jnp.minimum(2 * i + 1, V // hb))),
      ],
      out_specs=pl.BlockSpec((hb, 128), lambda i: (i, 0)),
      out_shape=jax.ShapeDtypeStruct((n_blk * hb, 128), jnp.int32),
  ), n_blk * _SB


def _sc_gather(n_tok, Vp, chunk):
  tpw = n_tok // _NW
  n_chunks = tpw // chunk
  nsub = chunk // 128
  mesh = plsc.VectorSubcoreMesh(core_axis_name="c", subcore_axis_name="s")

  @functools.partial(
      pl.kernel,
      mesh=mesh,
      compiler_params=pltpu.CompilerParams(use_tc_tiling_on_sc=False),
      out_type=jax.ShapeDtypeStruct((n_tok, 64), jnp.int32),
      scratch_types=[
          pltpu.VMEM((nsub, 128), jnp.int32),
          pltpu.VMEM((nsub, 128), jnp.int32),
          pltpu.VMEM((chunk, 64), jnp.int32),
          pltpu.SemaphoreType.DMA,
      ],
  )
  def k(xf_hbm, t_hbm, g_hbm, idx_v, idx2_v, rows_v, sem):
    wid = lax.axis_index("s") * _NC + lax.axis_index("c")
    start = wid * tpw

    def body(ci, carry):
      off = start + ci * chunk
      pltpu.sync_copy(xf_hbm.at[pl.ds(off // 128, nsub)], idx_v)
      for j in range(nsub):
        for kk in range(8):
          sl = pl.ds(kk * 16, 16)
          v = idx_v[j, sl]
          idx2_v[j, sl] = (
              (v >> 15) << 15
          ) + ((v & 16383) << 1) + ((v >> 14) & 1)
      cps = [
          pltpu.async_copy(
              t_hbm.at[idx2_v.at[j]],
              rows_v.at[pl.ds(j * 128, 128)],
              sem,
          )
          for j in range(nsub)
      ]
      for cp in cps:
        cp.wait()
      pltpu.sync_copy(rows_v, g_hbm.at[pl.ds(off, chunk)])
      return carry

    lax.fori_loop(0, n_chunks, body, 0)

  return k


def _tc_combine(n_tok, L, D, bc):
  B = n_tok // L
  nb = B // bc
  hc = bc // 2

  def body(g_ref, mlo_ref, mhi_ref, out_ref):
    dims = (((1,), (1,)), ((), ()))
    for t in range(2):
      gu = lax.bitcast_convert_type(
          g_ref[pl.ds(t * hc, hc), :], jnp.uint32)
      lo = lax.bitcast_convert_type(
          (gu & 0xFFFF).astype(jnp.uint16), jnp.bfloat16)
      hi = lax.bitcast_convert_type(
          (gu >> 16).astype(jnp.uint16), jnp.bfloat16)
      out_ref[pl.ds(t * D, D), :hc] = lax.dot_general(
          mlo_ref[...], lo[:, :64], dims, preferred_element_type=jnp.float32
      ) + lax.dot_general(
          mhi_ref[...], hi[:, :64], dims, preferred_element_type=jnp.float32)
      out_ref[pl.ds(t * D, D), hc:] = lax.dot_general(
          mlo_ref[...], lo[:, 64:], dims, preferred_element_type=jnp.float32
      ) + lax.dot_general(
          mhi_ref[...], hi[:, 64:], dims, preferred_element_type=jnp.float32)

  return pl.pallas_call(
      body,
      grid=(L // 2,),
      in_specs=[
          pl.BlockSpec((2 * hc, 128), lambda i: (i, 0)),
          pl.BlockSpec((D, D), lambda i: (0, 0)),
          pl.BlockSpec((D, D), lambda i: (0, 0)),
      ],
      out_specs=pl.BlockSpec((2 * D, bc), lambda i: (i, 0)),
      out_shape=jax.ShapeDtypeStruct((L * D, B), jnp.float32),
  )


@jax.jit
def kernel(x, weight, lora_a, lora_b):
  B, L = x.shape
  V, D = weight.shape
  R = lora_a.shape[0]
  n_tok = B * L
  bc = 4096
  hc = bc // 2

  prep, Vp = _tc_prep(V, D, R)
  wT = weight.T
  table = prep(wT, lora_a, wT, lora_a)
  t64 = table.reshape(Vp, 64)

  xp = (x.T.astype(jnp.int32)
        .reshape(L, B // bc, 2, hc)
        .transpose(0, 1, 3, 2)
        .reshape(n_tok // 128, 128))
  g = _sc_gather(n_tok, Vp, chunk=1280)(xp, t64)
  g2 = g.reshape(n_tok // 2, 128)

  mlo = jnp.eye(D, dtype=jnp.bfloat16)
  mhi = jnp.concatenate(
      [lora_b, jnp.zeros((D, D - R), jnp.float32)], axis=1).astype(jnp.bfloat16)
  out2 = _tc_combine(n_tok, L, D, bc=bc)(g2, mlo, mhi)
  return out2.reshape(L, D, B).transpose(2, 0, 1)
